# Initial kernel scaffold; baseline (speedup 1.0000x reference)
import functools

import jax
import jax.numpy as jnp
import numpy as np
from jax import lax
from jax.experimental import pallas as pl
from jax.experimental.pallas import tpu as pltpu
from jax.experimental.pallas import tpu_sc as plsc

_N = 10000
_E = 320000
_D = 128
_NC = 2
_NS = 16
_EPT = _E // (_NC * _NS)
_CH = 128
_NFULL = _EPT // _CH
_TAIL = _EPT - _NFULL * _CH
_RPT = _N // _NS
_DEGR = 10304
_DRPT = _DEGR // _NS

_TAPS = {0: [(0, 1), (-1, 3)], 1: [(1, 0), (0, 2)]}

_mesh = plsc.VectorSubcoreMesh(core_axis_name="c", subcore_axis_name="s")


@functools.partial(
    pl.kernel,
    out_type=jax.ShapeDtypeStruct((_NC, _DEGR, 16), jnp.float32),
    mesh=_mesh,
    scratch_types=[
        pltpu.VMEM_SHARED((_DEGR, 16), jnp.float32),
        pltpu.VMEM((_CH, 16), jnp.float32),
        pltpu.VMEM((_CH,), jnp.int32),
        pltpu.VMEM((_TAIL,), jnp.int32),
    ],
)
def _sc_deg(dst_hbm, zeros_hbm, out_hbm, table, obuf, didx, didx16):
    c = lax.axis_index("c")
    s = lax.axis_index("s")
    r0 = s * _DRPT
    pltpu.sync_copy(zeros_hbm.at[pl.ds(r0, _DRPT)], table.at[pl.ds(r0, _DRPT)])
    e0 = jnp.where(lax.iota(jnp.int32, 16) == 0, 1.0, 0.0)
    for r in range(_CH):
        obuf[r, :] = e0
    plsc.subcore_barrier()
    ebase = (c * _NS + s) * _EPT

    @pl.loop(0, _NFULL)
    def _chunk(j):
        base = ebase + j * _CH
        pltpu.sync_copy(dst_hbm.at[pl.ds(base, _CH)], didx)
        pltpu.sync_copy(obuf, table.at[didx], add=True)

    base = ebase + _NFULL * _CH
    pltpu.sync_copy(dst_hbm.at[pl.ds(base, _TAIL)], didx16)
    pltpu.sync_copy(obuf.at[pl.ds(0, _TAIL)], table.at[didx16], add=True)
    plsc.subcore_barrier()
    pltpu.sync_copy(table.at[pl.ds(r0, _DRPT)], out_hbm.at[c, pl.ds(r0, _DRPT)])


@functools.partial(
    pl.kernel,
    out_type=jax.ShapeDtypeStruct((_NC, _N, _D), jnp.float32),
    mesh=_mesh,
    scratch_types=[
        pltpu.VMEM_SHARED((_N, _D), jnp.float32),
        pltpu.VMEM((_CH, _D), jnp.float32),
        pltpu.VMEM((_CH,), jnp.int32),
        pltpu.VMEM((_CH,), jnp.int32),
        pltpu.VMEM((_TAIL, _D), jnp.float32),
        pltpu.VMEM((_TAIL,), jnp.int32),
        pltpu.VMEM((_TAIL,), jnp.int32),
        pltpu.SemaphoreType.DMA,
    ],
)
def _sc_scatter(g_hbm, zeros_hbm, src_hbm, dst_hbm, out_hbm,
                accum, gbuf, sidx, didx, gbuf16, sidx16, didx16, sem):
    c = lax.axis_index("c")
    s = lax.axis_index("s")
    r0 = s * _RPT

    @pl.when(c == 0)
    def _():
        pltpu.sync_copy(g_hbm.at[pl.ds(r0, _RPT)], accum.at[pl.ds(r0, _RPT)])

    @pl.when(c != 0)
    def _():
        pltpu.sync_copy(zeros_hbm.at[pl.ds(r0, _RPT)], accum.at[pl.ds(r0, _RPT)])

    plsc.subcore_barrier()
    ebase = (c * _NS + s) * _EPT

    @pl.loop(0, _NFULL)
    def _chunk(j):
        base = ebase + j * _CH
        pltpu.sync_copy(src_hbm.at[pl.ds(base, _CH)], sidx)
        pltpu.sync_copy(dst_hbm.at[pl.ds(base, _CH)], didx)
        pltpu.async_copy(g_hbm.at[sidx], gbuf, sem).wait()
        pltpu.sync_copy(gbuf, accum.at[didx], add=True)

    base = ebase + _NFULL * _CH
    pltpu.sync_copy(src_hbm.at[pl.ds(base, _TAIL)], sidx16)
    pltpu.sync_copy(dst_hbm.at[pl.ds(base, _TAIL)], didx16)
    pltpu.async_copy(g_hbm.at[sidx16], gbuf16, sem).wait()
    pltpu.sync_copy(gbuf16, accum.at[didx16], add=True)
    plsc.subcore_barrier()
    pltpu.sync_copy(accum.at[pl.ds(r0, _RPT)], out_hbm.at[c, pl.ds(r0, _RPT)])


def _dinv_body(dp_ref, o_ref):
    o_ref[...] = lax.rsqrt(dp_ref[0] + dp_ref[1] + 1.0)


_tc_dinv = pl.pallas_call(
    _dinv_body,
    out_shape=jax.ShapeDtypeStruct((_DEGR // 128, 128), jnp.float32),
)


def _m1_body(x_ref, w_ref, dv_ref, o_ref):
    o_ref[...] = jnp.dot(x_ref[...], w_ref[...],
                         preferred_element_type=jnp.float32) * dv_ref[...]


_tc_m1 = pl.pallas_call(
    _m1_body,
    grid=(5,),
    in_specs=[
        pl.BlockSpec((2000, _D), lambda i: (i, 0)),
        pl.BlockSpec((_D, _D), lambda i: (0, 0)),
        pl.BlockSpec((2000, 1), lambda i: (i, 0)),
    ],
    out_specs=pl.BlockSpec((2000, _D), lambda i: (i, 0)),
    out_shape=jax.ShapeDtypeStruct((_N, _D), jnp.float32),
)


def _m_body(p_ref, dv_ref, b_ref, w_ref, o_ref):
    a = jnp.maximum((p_ref[0] + p_ref[1]) * dv_ref[...] + b_ref[...], 0.0)
    o_ref[...] = jnp.dot(a, w_ref[...],
                         preferred_element_type=jnp.float32) * dv_ref[...]


_tc_m = pl.pallas_call(
    _m_body,
    grid=(5,),
    in_specs=[
        pl.BlockSpec((2, 2000, _D), lambda i: (0, i, 0)),
        pl.BlockSpec((2000, 1), lambda i: (i, 0)),
        pl.BlockSpec((1, _D), lambda i: (0, 0)),
        pl.BlockSpec((_D, _D), lambda i: (0, 0)),
    ],
    out_specs=pl.BlockSpec((2000, _D), lambda i: (i, 0)),
    out_shape=jax.ShapeDtypeStruct((_N, _D), jnp.float32),
)


def _f_body(p_ref, dv_ref, b3_ref, bi_ref, wl_ref, bl_ref, wdr_ref, bdr_ref,
            lat_ref, hdt_ref):
    h3 = jnp.maximum((p_ref[0] + p_ref[1]) * dv_ref[...] + b3_ref[...], 0.0)
    oh = (lax.broadcasted_iota(jnp.int32, (64, _N), 0) == bi_ref[...]
          ).astype(jnp.float32)
    cnts = jnp.sum(oh, axis=1, keepdims=True)
    pooled = jnp.dot(oh, h3, preferred_element_type=jnp.float32) \
        / jnp.maximum(cnts, 1.0)
    latent = jnp.dot(pooled, wl_ref[...],
                     preferred_element_type=jnp.float32) + bl_ref[...]
    lat_ref[...] = latent
    hdt_ref[...] = jnp.dot(latent, wdr_ref[...],
                           preferred_element_type=jnp.float32) + bdr_ref[...]


_tc_f = pl.pallas_call(
    _f_body,
    out_shape=(
        jax.ShapeDtypeStruct((64, 256), jnp.float32),
        jax.ShapeDtypeStruct((64, 4096), jnp.float32),
    ),
    compiler_params=pltpu.CompilerParams(vmem_limit_bytes=100 * 1024 * 1024),
)


def _c1_body(a_ref, b_ref, bias_ref, o_ref):
    for rd in range(2):
        for rh in range(2):
            acc = None
            for tdi in range(2):
                for thi in range(2):
                    c = ((rd * 2 + tdi) * 2 + rh) * 2 + thi
                    t = jnp.dot(a_ref[c], b_ref[c],
                                preferred_element_type=jnp.float32)
                    acc = t if acc is None else acc + t
            o_ref[rd, rh] = jnp.maximum(acc + bias_ref[...], 0.0)


_tc_c1 = pl.pallas_call(
    _c1_body,
    out_shape=jax.ShapeDtypeStruct((2, 2, 1024, 256), jnp.float32),
    compiler_params=pltpu.CompilerParams(vmem_limit_bytes=100 * 1024 * 1024),
)


def _c2_body(a_ref, b_ref, bias_ref, o_ref):
    for rd in range(2):
        acc = None
        for tdi in range(2):
            c = rd * 2 + tdi
            t = jnp.dot(a_ref[c], b_ref[c], preferred_element_type=jnp.float32)
            acc = t if acc is None else acc + t
        v = acc + bias_ref[...]
        o_ref[rd] = jnp.maximum(v, 0.0) + jnp.log(1.0 + jnp.exp(-jnp.abs(v)))


_tc_c2 = pl.pallas_call(
    _c2_body,
    out_shape=jax.ShapeDtypeStruct((2, 512, 256), jnp.float32),
    compiler_params=pltpu.CompilerParams(vmem_limit_bytes=120 * 1024 * 1024),
)


def _band(I):
    m = np.zeros((4, I + 2, 2 * I), np.float32)
    for t in range(4):
        for p in range(1, I + 1):
            o = 2 * (p - 1) + t - 1
            if 0 <= o < 2 * I:
                m[t, p, o] = 1.0
    return m


_S1 = jnp.asarray(_band(4))
_S2 = jnp.asarray(_band(8))


def kernel(x, edge_index, batch_index, W1, b1, W2, b2, W3, b3, Wl, bl,
           Wd, bd, wdc1, bdc1, wdc2, bdc2):
    src = edge_index[0]
    dst = edge_index[1]
    zeros_deg = jnp.zeros((_DEGR, 16), jnp.float32)
    zeros_g = jnp.zeros((_N, _D), jnp.float32)

    degp = _sc_deg(dst, zeros_deg)
    dp = degp[:, :, 0].reshape(2, _DEGR // 128, 128)
    dinv2d = _tc_dinv(dp)
    dinv = dinv2d.reshape(_DEGR)[:_N][:, None]

    g1 = _tc_m1(x, W1, dinv)
    s1 = _sc_scatter(g1, zeros_g, src, dst)
    g2 = _tc_m(s1, dinv, b1[None, :], W2)
    s2 = _sc_scatter(g2, zeros_g, src, dst)
    g3 = _tc_m(s2, dinv, b2[None, :], W3)
    s3 = _sc_scatter(g3, zeros_g, src, dst)

    Wdr = Wd.reshape(256, 64, 64).transpose(0, 2, 1).reshape(256, 4096)
    bdr = bd.reshape(64, 64).T.reshape(4096)
    latent, hdt = _tc_f(s3, dinv, b3[None, :], batch_index[None, :],
                        Wl, bl[None, :], Wdr, bdr[None, :])

    X = hdt.reshape(64, 4, 4, 4, 64)
    Xp = jnp.pad(X, ((0, 0), (1, 1), (1, 1), (1, 1), (0, 0)))
    a1list, b1list = [], []
    for rd in (0, 1):
        for (dd, td) in _TAPS[rd]:
            for rh in (0, 1):
                for (dh, th) in _TAPS[rh]:
                    a1list.append(
                        Xp[:, 1 + dd:5 + dd, 1 + dh:5 + dh, :, :]
                        .reshape(1024, 384))
                    b1list.append(
                        jnp.einsum("tpo,cdt->pcod", _S1, wdc1[:, :, td, th, :])
                        .reshape(384, 256))
    A1 = jnp.stack(a1list)
    B1 = jnp.stack(b1list)
    o1 = _tc_c1(A1, B1, jnp.tile(bdc1, 8)[None, :])

    X2 = (o1.reshape(2, 2, 64, 4, 4, 8, 32)
          .transpose(2, 3, 0, 4, 1, 5, 6).reshape(64, 8, 8, 8, 32))
    X2p = jnp.pad(X2, ((0, 0), (1, 1), (1, 1), (1, 1), (0, 0)))
    a2list, b2list = [], []
    for rd in (0, 1):
        for (dd, td) in _TAPS[rd]:
            a2list.append(X2p[:, 1 + dd:9 + dd, :, :, :].reshape(512, 3200))
            b2list.append(
                jnp.einsum("tpo,uqr,ctu->pqcor", _S2, _S2, wdc2[:, 0, td, :, :])
                .reshape(3200, 256))
    A2 = jnp.stack(a2list)
    B2 = jnp.stack(b2list)
    o2 = _tc_c2(A2, B2, bdc2[None, :])

    voxels = (o2.reshape(2, 64, 8, 16, 16)
              .transpose(1, 2, 0, 3, 4).reshape(64, 16, 16, 16))[:, None]
    return voxels, latent


# speedup vs baseline: 9.1451x; 9.1451x over previous
import functools

import jax
import jax.numpy as jnp
import numpy as np
from jax import lax
from jax.experimental import pallas as pl
from jax.experimental.pallas import tpu as pltpu
from jax.experimental.pallas import tpu_sc as plsc

_N = 10000
_E = 320000
_D = 128
_NC = 2
_NS = 16
_EPT = _E // (_NC * _NS)
_CH = 128
_NFULL = _EPT // _CH
_TAIL = _EPT - _NFULL * _CH
_RPT = 624
_DGR = 1280
_DGRPT = _DGR // _NS

_TAPS = {0: [(0, 1), (-1, 3)], 1: [(1, 0), (0, 2)]}


def _sc_deg_body(dst_hbm, zeros_hbm, out_hbm, table, obuf,
                 didx, didx3, didx16, didx316):
    c = lax.axis_index("c")
    s = lax.axis_index("s")
    r0 = s * _DGRPT
    pltpu.sync_copy(zeros_hbm.at[pl.ds(r0, _DGRPT)], table.at[pl.ds(r0, _DGRPT)])
    pltpu.sync_copy(zeros_hbm.at[pl.ds(0, _CH)], obuf)
    plsc.subcore_barrier()
    ones16 = jnp.full((16,), 1.0, jnp.float32)
    zeros16 = jnp.zeros((16,), jnp.float32)
    iota16 = lax.iota(jnp.int32, 16)
    ebase = (c * _NS + s) * _EPT

    @pl.loop(0, _NFULL)
    def _chunk(j):
        base = ebase + j * _CH
        pltpu.sync_copy(dst_hbm.at[pl.ds(base, _CH)], didx)
        for g in range(8):
            v = didx[pl.ds(g * 16, 16)]
            rows = iota16 + (g * 16)
            cols = (v & 7) * 16
            plsc.store_scatter(obuf, [rows, cols], ones16)
            didx3[pl.ds(g * 16, 16)] = lax.shift_right_logical(v, 3)
        pltpu.sync_copy(obuf, table.at[didx3], add=True)
        for g in range(8):
            v = didx[pl.ds(g * 16, 16)]
            rows = iota16 + (g * 16)
            cols = (v & 7) * 16
            plsc.store_scatter(obuf, [rows, cols], zeros16)

    base = ebase + _NFULL * _CH
    pltpu.sync_copy(dst_hbm.at[pl.ds(base, _TAIL)], didx16)
    v = didx16[...]
    cols = (v & 7) * 16
    plsc.store_scatter(obuf, [iota16, cols], ones16)
    didx316[...] = lax.shift_right_logical(v, 3)
    pltpu.sync_copy(obuf.at[pl.ds(0, _TAIL)], table.at[didx316], add=True)
    plsc.subcore_barrier()
    pltpu.sync_copy(table.at[pl.ds(r0, _DGRPT)], out_hbm.at[c, pl.ds(r0, _DGRPT)])


def _sc_scatter_body(g_hbm, zeros_hbm, src_hbm, dst_hbm, out_hbm,
                     accum, gbuf, sidx, didx, gbuf16, sidx16, didx16, sem):
    c = lax.axis_index("c")
    s = lax.axis_index("s")
    r0 = s * _RPT
    rtail = _NS * _RPT

    @pl.when(c == 0)
    def _():
        pltpu.sync_copy(g_hbm.at[pl.ds(r0, _RPT)], accum.at[pl.ds(r0, _RPT)])

        @pl.when(s == _NS - 1)
        def _():
            pltpu.sync_copy(g_hbm.at[pl.ds(rtail, _N - rtail)],
                            accum.at[pl.ds(rtail, _N - rtail)])

    @pl.when(c != 0)
    def _():
        pltpu.sync_copy(zeros_hbm.at[pl.ds(r0, _RPT)], accum.at[pl.ds(r0, _RPT)])

        @pl.when(s == _NS - 1)
        def _():
            pltpu.sync_copy(zeros_hbm.at[pl.ds(rtail, _N - rtail)],
                            accum.at[pl.ds(rtail, _N - rtail)])

    plsc.subcore_barrier()
    ebase = (c * _NS + s) * _EPT

    @pl.loop(0, _NFULL)
    def _chunk(j):
        base = ebase + j * _CH
        pltpu.sync_copy(src_hbm.at[pl.ds(base, _CH)], sidx)
        pltpu.sync_copy(dst_hbm.at[pl.ds(base, _CH)], didx)
        pltpu.async_copy(g_hbm.at[sidx], gbuf, sem).wait()
        pltpu.sync_copy(gbuf, accum.at[didx], add=True)

    base = ebase + _NFULL * _CH
    pltpu.sync_copy(src_hbm.at[pl.ds(base, _TAIL)], sidx16)
    pltpu.sync_copy(dst_hbm.at[pl.ds(base, _TAIL)], didx16)
    pltpu.async_copy(g_hbm.at[sidx16], gbuf16, sem).wait()
    pltpu.sync_copy(gbuf16, accum.at[didx16], add=True)
    plsc.subcore_barrier()
    pltpu.sync_copy(accum.at[pl.ds(r0, _RPT)], out_hbm.at[c, pl.ds(r0, _RPT)])

    @pl.when(s == _NS - 1)
    def _():
        pltpu.sync_copy(accum.at[pl.ds(rtail, _N - rtail)],
                        out_hbm.at[c, pl.ds(rtail, _N - rtail)])


@functools.cache
def _get_sc_kernels():
    mesh = plsc.VectorSubcoreMesh(core_axis_name="c", subcore_axis_name="s",
                                  num_cores=_NC, num_subcores=_NS)
    sc_deg = pl.kernel(
        _sc_deg_body,
        out_type=jax.ShapeDtypeStruct((_NC, _DGR, _D), jnp.float32),
        mesh=mesh,
        compiler_params=pltpu.CompilerParams(needs_layout_passes=False),
        scratch_types=[
            pltpu.VMEM_SHARED((_DGR, _D), jnp.float32),
            pltpu.VMEM((_CH, _D), jnp.float32),
            pltpu.VMEM((_CH,), jnp.int32),
            pltpu.VMEM((_CH,), jnp.int32),
            pltpu.VMEM((_TAIL,), jnp.int32),
            pltpu.VMEM((_TAIL,), jnp.int32),
        ],
    )
    sc_scatter = pl.kernel(
        _sc_scatter_body,
        out_type=jax.ShapeDtypeStruct((_NC, _N, _D), jnp.float32),
        mesh=mesh,
        scratch_types=[
            pltpu.VMEM_SHARED((_N, _D), jnp.float32),
            pltpu.VMEM((_CH, _D), jnp.float32),
            pltpu.VMEM((_CH,), jnp.int32),
            pltpu.VMEM((_CH,), jnp.int32),
            pltpu.VMEM((_TAIL, _D), jnp.float32),
            pltpu.VMEM((_TAIL,), jnp.int32),
            pltpu.VMEM((_TAIL,), jnp.int32),
            pltpu.SemaphoreType.DMA,
        ],
    )
    return sc_deg, sc_scatter


def _dinv_body(dp_ref, o_ref):
    o_ref[...] = lax.rsqrt(dp_ref[0] + dp_ref[1] + 1.0)


_tc_dinv = pl.pallas_call(
    _dinv_body,
    out_shape=jax.ShapeDtypeStruct((80, 128), jnp.float32),
)


def _m1_body(x_ref, w_ref, dv_ref, o_ref):
    o_ref[...] = jnp.dot(x_ref[...], w_ref[...],
                         preferred_element_type=jnp.float32) * dv_ref[...]


_tc_m1 = pl.pallas_call(
    _m1_body,
    grid=(5,),
    in_specs=[
        pl.BlockSpec((2000, _D), lambda i: (i, 0)),
        pl.BlockSpec((_D, _D), lambda i: (0, 0)),
        pl.BlockSpec((2000, 1), lambda i: (i, 0)),
    ],
    out_specs=pl.BlockSpec((2000, _D), lambda i: (i, 0)),
    out_shape=jax.ShapeDtypeStruct((_N, _D), jnp.float32),
)


def _m_body(p_ref, dv_ref, b_ref, w_ref, o_ref):
    a = jnp.maximum((p_ref[0] + p_ref[1]) * dv_ref[...] + b_ref[...], 0.0)
    o_ref[...] = jnp.dot(a, w_ref[...],
                         preferred_element_type=jnp.float32) * dv_ref[...]


_tc_m = pl.pallas_call(
    _m_body,
    grid=(5,),
    in_specs=[
        pl.BlockSpec((2, 2000, _D), lambda i: (0, i, 0)),
        pl.BlockSpec((2000, 1), lambda i: (i, 0)),
        pl.BlockSpec((1, _D), lambda i: (0, 0)),
        pl.BlockSpec((_D, _D), lambda i: (0, 0)),
    ],
    out_specs=pl.BlockSpec((2000, _D), lambda i: (i, 0)),
    out_shape=jax.ShapeDtypeStruct((_N, _D), jnp.float32),
)


def _f_body(p_ref, dv_ref, b3_ref, bi_ref, wl_ref, bl_ref, wdr_ref, bdr_ref,
            lat_ref, hdt_ref):
    h3 = jnp.maximum((p_ref[0] + p_ref[1]) * dv_ref[...] + b3_ref[...], 0.0)
    oh = (lax.broadcasted_iota(jnp.int32, (64, _N), 0) == bi_ref[...]
          ).astype(jnp.float32)
    cnts = jnp.sum(oh, axis=1, keepdims=True)
    pooled = jnp.dot(oh, h3, preferred_element_type=jnp.float32) \
        / jnp.maximum(cnts, 1.0)
    latent = jnp.dot(pooled, wl_ref[...],
                     preferred_element_type=jnp.float32) + bl_ref[...]
    lat_ref[...] = latent
    hdt_ref[...] = jnp.dot(latent, wdr_ref[...],
                           preferred_element_type=jnp.float32) + bdr_ref[...]


_tc_f = pl.pallas_call(
    _f_body,
    out_shape=(
        jax.ShapeDtypeStruct((64, 256), jnp.float32),
        jax.ShapeDtypeStruct((64, 4096), jnp.float32),
    ),
    compiler_params=pltpu.CompilerParams(vmem_limit_bytes=100 * 1024 * 1024),
)


def _c1_body(a_ref, b_ref, bias_ref, o_ref):
    for rd in range(2):
        for rh in range(2):
            acc = None
            for tdi in range(2):
                for thi in range(2):
                    c = ((rd * 2 + tdi) * 2 + rh) * 2 + thi
                    t = jnp.dot(a_ref[c], b_ref[c],
                                preferred_element_type=jnp.float32)
                    acc = t if acc is None else acc + t
            o_ref[rd, rh] = jnp.maximum(acc + bias_ref[...], 0.0)


_tc_c1 = pl.pallas_call(
    _c1_body,
    out_shape=jax.ShapeDtypeStruct((2, 2, 1024, 256), jnp.float32),
    compiler_params=pltpu.CompilerParams(vmem_limit_bytes=100 * 1024 * 1024),
)


def _c2_body(a_ref, b_ref, bias_ref, o_ref):
    for rd in range(2):
        acc = None
        for tdi in range(2):
            c = rd * 2 + tdi
            t = jnp.dot(a_ref[c], b_ref[c], preferred_element_type=jnp.float32)
            acc = t if acc is None else acc + t
        v = acc + bias_ref[...]
        o_ref[rd] = jnp.maximum(v, 0.0) + jnp.log(1.0 + jnp.exp(-jnp.abs(v)))


_tc_c2 = pl.pallas_call(
    _c2_body,
    out_shape=jax.ShapeDtypeStruct((2, 512, 256), jnp.float32),
    compiler_params=pltpu.CompilerParams(vmem_limit_bytes=120 * 1024 * 1024),
)


def _band(I):
    m = np.zeros((4, I + 2, 2 * I), np.float32)
    for t in range(4):
        for p in range(1, I + 1):
            o = 2 * (p - 1) + t - 1
            if 0 <= o < 2 * I:
                m[t, p, o] = 1.0
    return m


_S1 = _band(4)
_S2 = _band(8)


def kernel(x, edge_index, batch_index, W1, b1, W2, b2, W3, b3, Wl, bl,
           Wd, bd, wdc1, bdc1, wdc2, bdc2):
    src = edge_index[0]
    dst = edge_index[1]
    zeros_g = jnp.zeros((_N, _D), jnp.float32)
    _sc_deg, _sc_scatter = _get_sc_kernels()

    degp = _sc_deg(dst, zeros_g)
    dp = degp.reshape(2, _DGR, 8, 16)[:, :, :, 0].reshape(2, 80, 128)
    dinv2d = _tc_dinv(dp)
    dinv = dinv2d.reshape(10240)[:_N][:, None]

    g1 = _tc_m1(x, W1, dinv)
    s1 = _sc_scatter(g1, zeros_g, src, dst)
    g2 = _tc_m(s1, dinv, b1[None, :], W2)
    s2 = _sc_scatter(g2, zeros_g, src, dst)
    g3 = _tc_m(s2, dinv, b2[None, :], W3)
    s3 = _sc_scatter(g3, zeros_g, src, dst)

    Wdr = Wd.reshape(256, 64, 64).transpose(0, 2, 1).reshape(256, 4096)
    bdr = bd.reshape(64, 64).T.reshape(4096)
    latent, hdt = _tc_f(s3, dinv, b3[None, :], batch_index[None, :],
                        Wl, bl[None, :], Wdr, bdr[None, :])

    X = hdt.reshape(64, 4, 4, 4, 64)
    Xp = jnp.pad(X, ((0, 0), (1, 1), (1, 1), (1, 1), (0, 0)))
    a1list, b1list = [], []
    for rd in (0, 1):
        for (dd, td) in _TAPS[rd]:
            for rh in (0, 1):
                for (dh, th) in _TAPS[rh]:
                    a1list.append(
                        Xp[:, 1 + dd:5 + dd, 1 + dh:5 + dh, :, :]
                        .reshape(1024, 384))
                    b1list.append(
                        jnp.einsum("tpo,cdt->pcod", _S1, wdc1[:, :, td, th, :])
                        .reshape(384, 256))
    A1 = jnp.stack(a1list)
    B1 = jnp.stack(b1list)
    o1 = _tc_c1(A1, B1, jnp.tile(bdc1, 8)[None, :])

    X2 = (o1.reshape(2, 2, 64, 4, 4, 8, 32)
          .transpose(2, 3, 0, 4, 1, 5, 6).reshape(64, 8, 8, 8, 32))
    X2p = jnp.pad(X2, ((0, 0), (1, 1), (1, 1), (1, 1), (0, 0)))
    a2list, b2list = [], []
    for rd in (0, 1):
        for (dd, td) in _TAPS[rd]:
            a2list.append(X2p[:, 1 + dd:9 + dd, :, :, :].reshape(512, 3200))
            b2list.append(
                jnp.einsum("tpo,uqr,ctu->pqcor", _S2, _S2, wdc2[:, 0, td, :, :])
                .reshape(3200, 256))
    A2 = jnp.stack(a2list)
    B2 = jnp.stack(b2list)
    o2 = _tc_c2(A2, B2, bdc2[None, :])

    voxels = (o2.reshape(2, 64, 8, 16, 16)
              .transpose(1, 2, 0, 3, 4).reshape(64, 16, 16, 16))[:, None]
    return voxels, latent


# speedup vs baseline: 10.6360x; 1.1630x over previous
import functools

import jax
import jax.numpy as jnp
import numpy as np
from jax import lax
from jax.experimental import pallas as pl
from jax.experimental.pallas import tpu as pltpu
from jax.experimental.pallas import tpu_sc as plsc

_N = 10000
_E = 320000
_D = 128
_NC = 2
_NS = 16
_EPT = _E // (_NC * _NS)
_CH = 128
_NFULL = _EPT // _CH
_TAIL = _EPT - _NFULL * _CH
_RPT = 624
_DGR = 1280
_DGRPT = _DGR // _NS

_TAPS = {0: [(0, 1), (-1, 3)], 1: [(1, 0), (0, 2)]}


def _sc_deg_body(dst_hbm, zeros_hbm, out_hbm, table, obuf,
                 didx, didx3, didx16, didx316):
    c = lax.axis_index("c")
    s = lax.axis_index("s")
    r0 = s * _DGRPT
    pltpu.sync_copy(zeros_hbm.at[pl.ds(r0, _DGRPT)], table.at[pl.ds(r0, _DGRPT)])
    pltpu.sync_copy(zeros_hbm.at[pl.ds(0, _CH)], obuf)
    plsc.subcore_barrier()
    ones16 = jnp.full((16,), 1.0, jnp.float32)
    zeros16 = jnp.zeros((16,), jnp.float32)
    iota16 = lax.iota(jnp.int32, 16)
    ebase = (c * _NS + s) * _EPT

    @pl.loop(0, _NFULL)
    def _chunk(j):
        base = ebase + j * _CH
        pltpu.sync_copy(dst_hbm.at[pl.ds(base, _CH)], didx)
        for g in range(8):
            v = didx[pl.ds(g * 16, 16)]
            rows = iota16 + (g * 16)
            cols = (v & 7) * 16
            plsc.store_scatter(obuf, [rows, cols], ones16)
            didx3[pl.ds(g * 16, 16)] = lax.shift_right_logical(v, 3)
        pltpu.sync_copy(obuf, table.at[didx3], add=True)
        for g in range(8):
            v = didx[pl.ds(g * 16, 16)]
            rows = iota16 + (g * 16)
            cols = (v & 7) * 16
            plsc.store_scatter(obuf, [rows, cols], zeros16)

    base = ebase + _NFULL * _CH
    pltpu.sync_copy(dst_hbm.at[pl.ds(base, _TAIL)], didx16)
    v = didx16[...]
    cols = (v & 7) * 16
    plsc.store_scatter(obuf, [iota16, cols], ones16)
    didx316[...] = lax.shift_right_logical(v, 3)
    pltpu.sync_copy(obuf.at[pl.ds(0, _TAIL)], table.at[didx316], add=True)
    plsc.subcore_barrier()
    pltpu.sync_copy(table.at[pl.ds(r0, _DGRPT)], out_hbm.at[c, pl.ds(r0, _DGRPT)])


def _sc_scatter_body(g_hbm, zeros_hbm, src_hbm, dst_hbm, out_hbm,
                     accum, gbufa, gbufb, sidxa, sidxb, didxa, didxb,
                     gbuf16, sidx16, didx16, sema, semb):
    c = lax.axis_index("c")
    s = lax.axis_index("s")
    r0 = s * _RPT
    rtail = _NS * _RPT

    @pl.when(c == 0)
    def _():
        pltpu.sync_copy(g_hbm.at[pl.ds(r0, _RPT)], accum.at[pl.ds(r0, _RPT)])

        @pl.when(s == _NS - 1)
        def _():
            pltpu.sync_copy(g_hbm.at[pl.ds(rtail, _N - rtail)],
                            accum.at[pl.ds(rtail, _N - rtail)])

    @pl.when(c != 0)
    def _():
        pltpu.sync_copy(zeros_hbm.at[pl.ds(r0, _RPT)], accum.at[pl.ds(r0, _RPT)])

        @pl.when(s == _NS - 1)
        def _():
            pltpu.sync_copy(zeros_hbm.at[pl.ds(rtail, _N - rtail)],
                            accum.at[pl.ds(rtail, _N - rtail)])

    plsc.subcore_barrier()
    ebase = (c * _NS + s) * _EPT

    def _load_idx(chunk, si, di):
        base = ebase + chunk * _CH
        pltpu.sync_copy(src_hbm.at[pl.ds(base, _CH)], si)
        pltpu.sync_copy(dst_hbm.at[pl.ds(base, _CH)], di)

    _load_idx(0, sidxa, didxa)
    pltpu.async_copy(g_hbm.at[sidxa], gbufa, sema)

    @pl.loop(0, _NFULL // 2)
    def _pair(j):
        _load_idx(2 * j + 1, sidxb, didxb)
        pltpu.async_copy(g_hbm.at[sidxb], gbufb, semb)
        pltpu.make_async_copy(g_hbm.at[sidxa], gbufa, sema).wait()
        pltpu.sync_copy(gbufa, accum.at[didxa], add=True)

        @pl.when(j < _NFULL // 2 - 1)
        def _():
            _load_idx(2 * j + 2, sidxa, didxa)
            pltpu.async_copy(g_hbm.at[sidxa], gbufa, sema)

        pltpu.make_async_copy(g_hbm.at[sidxb], gbufb, semb).wait()
        pltpu.sync_copy(gbufb, accum.at[didxb], add=True)

    base = ebase + _NFULL * _CH
    pltpu.sync_copy(src_hbm.at[pl.ds(base, _TAIL)], sidx16)
    pltpu.sync_copy(dst_hbm.at[pl.ds(base, _TAIL)], didx16)
    pltpu.async_copy(g_hbm.at[sidx16], gbuf16, sema).wait()
    pltpu.sync_copy(gbuf16, accum.at[didx16], add=True)
    plsc.subcore_barrier()
    pltpu.sync_copy(accum.at[pl.ds(r0, _RPT)], out_hbm.at[c, pl.ds(r0, _RPT)])

    @pl.when(s == _NS - 1)
    def _():
        pltpu.sync_copy(accum.at[pl.ds(rtail, _N - rtail)],
                        out_hbm.at[c, pl.ds(rtail, _N - rtail)])


@functools.cache
def _get_sc_kernels():
    mesh = plsc.VectorSubcoreMesh(core_axis_name="c", subcore_axis_name="s",
                                  num_cores=_NC, num_subcores=_NS)
    sc_deg = pl.kernel(
        _sc_deg_body,
        out_type=jax.ShapeDtypeStruct((_NC, _DGR, _D), jnp.float32),
        mesh=mesh,
        compiler_params=pltpu.CompilerParams(needs_layout_passes=False),
        scratch_types=[
            pltpu.VMEM_SHARED((_DGR, _D), jnp.float32),
            pltpu.VMEM((_CH, _D), jnp.float32),
            pltpu.VMEM((_CH,), jnp.int32),
            pltpu.VMEM((_CH,), jnp.int32),
            pltpu.VMEM((_TAIL,), jnp.int32),
            pltpu.VMEM((_TAIL,), jnp.int32),
        ],
    )
    sc_scatter = pl.kernel(
        _sc_scatter_body,
        out_type=jax.ShapeDtypeStruct((_NC, _N, _D), jnp.float32),
        mesh=mesh,
        scratch_types=[
            pltpu.VMEM_SHARED((_N, _D), jnp.float32),
            pltpu.VMEM((_CH, _D), jnp.float32),
            pltpu.VMEM((_CH, _D), jnp.float32),
            pltpu.VMEM((_CH,), jnp.int32),
            pltpu.VMEM((_CH,), jnp.int32),
            pltpu.VMEM((_CH,), jnp.int32),
            pltpu.VMEM((_CH,), jnp.int32),
            pltpu.VMEM((_TAIL, _D), jnp.float32),
            pltpu.VMEM((_TAIL,), jnp.int32),
            pltpu.VMEM((_TAIL,), jnp.int32),
            pltpu.SemaphoreType.DMA,
            pltpu.SemaphoreType.DMA,
        ],
    )
    return sc_deg, sc_scatter


def _dinv_body(dp_ref, o_ref):
    o_ref[...] = lax.rsqrt(dp_ref[0] + dp_ref[1] + 1.0)


_tc_dinv = pl.pallas_call(
    _dinv_body,
    out_shape=jax.ShapeDtypeStruct((80, 128), jnp.float32),
)


def _m1a_body(x_ref, w_ref, o_ref):
    o_ref[...] = jnp.dot(x_ref[...], w_ref[...],
                         preferred_element_type=jnp.float32)


_tc_m1a = pl.pallas_call(
    _m1a_body,
    grid=(5,),
    in_specs=[
        pl.BlockSpec((2000, _D), lambda i: (i, 0)),
        pl.BlockSpec((_D, _D), lambda i: (0, 0)),
    ],
    out_specs=pl.BlockSpec((2000, _D), lambda i: (i, 0)),
    out_shape=jax.ShapeDtypeStruct((_N, _D), jnp.float32),
)


def _scale_body(y_ref, dv_ref, o_ref):
    o_ref[...] = y_ref[...] * dv_ref[...]


_tc_scale = pl.pallas_call(
    _scale_body,
    grid=(5,),
    in_specs=[
        pl.BlockSpec((2000, _D), lambda i: (i, 0)),
        pl.BlockSpec((2000, 1), lambda i: (i, 0)),
    ],
    out_specs=pl.BlockSpec((2000, _D), lambda i: (i, 0)),
    out_shape=jax.ShapeDtypeStruct((_N, _D), jnp.float32),
)


def _m_body(p_ref, dv_ref, b_ref, w_ref, o_ref):
    a = jnp.maximum((p_ref[0] + p_ref[1]) * dv_ref[...] + b_ref[...], 0.0)
    o_ref[...] = jnp.dot(a, w_ref[...],
                         preferred_element_type=jnp.float32) * dv_ref[...]


_tc_m = pl.pallas_call(
    _m_body,
    grid=(5,),
    in_specs=[
        pl.BlockSpec((2, 2000, _D), lambda i: (0, i, 0)),
        pl.BlockSpec((2000, 1), lambda i: (i, 0)),
        pl.BlockSpec((1, _D), lambda i: (0, 0)),
        pl.BlockSpec((_D, _D), lambda i: (0, 0)),
    ],
    out_specs=pl.BlockSpec((2000, _D), lambda i: (i, 0)),
    out_shape=jax.ShapeDtypeStruct((_N, _D), jnp.float32),
)


def _f_body(p_ref, dv_ref, b3_ref, bi_ref, wl_ref, bl_ref, wdr_ref, bdr_ref,
            lat_ref, hdt_ref):
    h3 = jnp.maximum((p_ref[0] + p_ref[1]) * dv_ref[...] + b3_ref[...], 0.0)
    oh = (lax.broadcasted_iota(jnp.int32, (64, _N), 0) == bi_ref[...]
          ).astype(jnp.float32)
    cnts = jnp.sum(oh, axis=1, keepdims=True)
    pooled = jnp.dot(oh, h3, preferred_element_type=jnp.float32) \
        / jnp.maximum(cnts, 1.0)
    latent = jnp.dot(pooled, wl_ref[...],
                     preferred_element_type=jnp.float32) + bl_ref[...]
    lat_ref[...] = latent
    hdt_ref[...] = jnp.dot(latent, wdr_ref[...],
                           preferred_element_type=jnp.float32) + bdr_ref[...]


_tc_f = pl.pallas_call(
    _f_body,
    out_shape=(
        jax.ShapeDtypeStruct((64, 256), jnp.float32),
        jax.ShapeDtypeStruct((64, 4096), jnp.float32),
    ),
    compiler_params=pltpu.CompilerParams(vmem_limit_bytes=100 * 1024 * 1024),
)


def _c1_body(a_ref, b_ref, bias_ref, o_ref):
    for rd in range(2):
        for rh in range(2):
            acc = None
            for tdi in range(2):
                for thi in range(2):
                    c = ((rd * 2 + tdi) * 2 + rh) * 2 + thi
                    t = jnp.dot(a_ref[c], b_ref[c],
                                preferred_element_type=jnp.float32)
                    acc = t if acc is None else acc + t
            o_ref[rd, rh] = jnp.maximum(acc + bias_ref[...], 0.0)


_tc_c1 = pl.pallas_call(
    _c1_body,
    out_shape=jax.ShapeDtypeStruct((2, 2, 1024, 256), jnp.float32),
    compiler_params=pltpu.CompilerParams(vmem_limit_bytes=100 * 1024 * 1024),
)


def _c2_body(a_ref, b_ref, bias_ref, o_ref):
    for rd in range(2):
        acc = None
        for tdi in range(2):
            c = rd * 2 + tdi
            t = jnp.dot(a_ref[c], b_ref[c], preferred_element_type=jnp.float32)
            acc = t if acc is None else acc + t
        v = acc + bias_ref[...]
        o_ref[rd] = jnp.maximum(v, 0.0) + jnp.log(1.0 + jnp.exp(-jnp.abs(v)))


_tc_c2 = pl.pallas_call(
    _c2_body,
    out_shape=jax.ShapeDtypeStruct((2, 512, 256), jnp.float32),
    compiler_params=pltpu.CompilerParams(vmem_limit_bytes=120 * 1024 * 1024),
)


def _band(I):
    m = np.zeros((4, I + 2, 2 * I), np.float32)
    for t in range(4):
        for p in range(1, I + 1):
            o = 2 * (p - 1) + t - 1
            if 0 <= o < 2 * I:
                m[t, p, o] = 1.0
    return m


_S1 = _band(4)
_S2 = _band(8)


def kernel(x, edge_index, batch_index, W1, b1, W2, b2, W3, b3, Wl, bl,
           Wd, bd, wdc1, bdc1, wdc2, bdc2):
    src = edge_index[0]
    dst = edge_index[1]
    zeros_g = jnp.zeros((_N, _D), jnp.float32)
    _sc_deg, _sc_scatter = _get_sc_kernels()

    y1 = _tc_m1a(x, W1)
    degp = _sc_deg(dst, zeros_g)
    dp = degp.reshape(2, _DGR, 8, 16)[:, :, :, 0].reshape(2, 80, 128)
    dinv2d = _tc_dinv(dp)
    dinv = dinv2d.reshape(10240)[:_N][:, None]

    g1 = _tc_scale(y1, dinv)
    s1 = _sc_scatter(g1, zeros_g, src, dst)
    g2 = _tc_m(s1, dinv, b1[None, :], W2)
    s2 = _sc_scatter(g2, zeros_g, src, dst)
    g3 = _tc_m(s2, dinv, b2[None, :], W3)
    s3 = _sc_scatter(g3, zeros_g, src, dst)

    Wdr = Wd.reshape(256, 64, 64).transpose(0, 2, 1).reshape(256, 4096)
    bdr = bd.reshape(64, 64).T.reshape(4096)
    latent, hdt = _tc_f(s3, dinv, b3[None, :], batch_index[None, :],
                        Wl, bl[None, :], Wdr, bdr[None, :])

    X = hdt.reshape(64, 4, 4, 4, 64)
    Xp = jnp.pad(X, ((0, 0), (1, 1), (1, 1), (1, 1), (0, 0)))
    a1list, b1list = [], []
    for rd in (0, 1):
        for (dd, td) in _TAPS[rd]:
            for rh in (0, 1):
                for (dh, th) in _TAPS[rh]:
                    a1list.append(
                        Xp[:, 1 + dd:5 + dd, 1 + dh:5 + dh, :, :]
                        .reshape(1024, 384))
                    b1list.append(
                        jnp.einsum("tpo,cdt->pcod", _S1, wdc1[:, :, td, th, :])
                        .reshape(384, 256))
    A1 = jnp.stack(a1list)
    B1 = jnp.stack(b1list)
    o1 = _tc_c1(A1, B1, jnp.tile(bdc1, 8)[None, :])

    X2 = (o1.reshape(2, 2, 64, 4, 4, 8, 32)
          .transpose(2, 3, 0, 4, 1, 5, 6).reshape(64, 8, 8, 8, 32))
    X2p = jnp.pad(X2, ((0, 0), (1, 1), (1, 1), (1, 1), (0, 0)))
    a2list, b2list = [], []
    for rd in (0, 1):
        for (dd, td) in _TAPS[rd]:
            a2list.append(X2p[:, 1 + dd:9 + dd, :, :, :].reshape(512, 3200))
            b2list.append(
                jnp.einsum("tpo,uqr,ctu->pqcor", _S2, _S2, wdc2[:, 0, td, :, :])
                .reshape(3200, 256))
    A2 = jnp.stack(a2list)
    B2 = jnp.stack(b2list)
    o2 = _tc_c2(A2, B2, bdc2[None, :])

    voxels = (o2.reshape(2, 64, 8, 16, 16)
              .transpose(1, 2, 0, 3, 4).reshape(64, 16, 16, 16))[:, None]
    return voxels, latent
